# Initial kernel scaffold; baseline (speedup 1.0000x reference)
import jax
import jax.numpy as jnp
from jax.experimental import pallas as pl


def kernel(pos, edge_index, batch, W0, b0, g0, be0, W1, b1, g1, be1, W2, b2, g2, be2, W3, b3, g3, be3, W4, b4, g4, be4, fW0, fb0, fg0, fbe0, fW1, fb1, fg1, fbe1, fW2, fb2):
    raise NotImplementedError("write your pallas kernel here")


# speedup vs baseline: 4.6397x; 4.6397x over previous
import functools

import jax
import jax.numpy as jnp
from jax import lax
from jax.experimental import pallas as pl
from jax.experimental.pallas import tpu as pltpu
from jax.experimental.pallas import tpu_sc as plsc

N = 10000
E = 160000
G = 32
CLASSES = 40

NC = 2
NS = 16
NW = NC * NS
CHUNK = 128
J = 40
E_PAD = NW * J * CHUNK
N_PAD = NS * 632
RPT = N_PAD // NS
BLK = 1000
NB = N // BLK


def _make_agg_kernel(F):
    mesh = plsc.VectorSubcoreMesh(core_axis_name="c", subcore_axis_name="s",
                                  num_cores=NC, num_subcores=NS)

    @functools.partial(
        pl.kernel, mesh=mesh,
        compiler_params=pltpu.CompilerParams(use_tc_tiling_on_sc=False),
        out_type=jax.ShapeDtypeStruct((NC, N_PAD, F), jnp.float32),
        scratch_types=[
            pltpu.VMEM((J, CHUNK), jnp.int32),
            pltpu.VMEM((J, CHUNK), jnp.int32),
            pltpu.VMEM((CHUNK, F), jnp.float32),
            pltpu.VMEM_SHARED((N_PAD, F), jnp.float32),
        ],
    )
    def agg(table_hbm, s_hbm, d_hbm, zeros_hbm, out_hbm, s_v, d_v, rows_v, acc_sh):
        cid = lax.axis_index("c")
        sid = lax.axis_index("s")
        wid = sid * NC + cid
        pltpu.sync_copy(zeros_hbm, acc_sh.at[pl.ds(sid * RPT, RPT)])
        pltpu.sync_copy(s_hbm.at[pl.ds(wid * J, J)], s_v)
        pltpu.sync_copy(d_hbm.at[pl.ds(wid * J, J)], d_v)
        plsc.subcore_barrier()

        def body(j, carry):
            pltpu.sync_copy(table_hbm.at[s_v.at[j]], rows_v)
            pltpu.sync_copy(rows_v, acc_sh.at[d_v.at[j]], add=True)
            return carry

        lax.fori_loop(0, J, body, 0)
        plsc.subcore_barrier()
        pltpu.sync_copy(acc_sh.at[pl.ds(sid * RPT, RPT)],
                        out_hbm.at[cid, pl.ds(sid * RPT, RPT)])

    return agg


_AGG = {F: _make_agg_kernel(F) for F in (16, 128)}


def _prep_body(h0, h1, pos16, dinv_ref, y0_ref):
    deg = h0[:, 0:1] + h1[:, 0:1] + 1.0
    dv = lax.rsqrt(deg)
    dinv_ref[...] = dv
    y0_ref[...] = pos16[...] * dv


def _prep(h0, h1, pos16):
    return pl.pallas_call(
        _prep_body,
        out_shape=[jax.ShapeDtypeStruct((N, 1), jnp.float32),
                   jax.ShapeDtypeStruct((N, 16), jnp.float32)],
    )(h0, h1, pos16)


def _mm_stats_body(q0, q1, xp, dinv, w, b, z_ref, st_ref, acc):
    i = pl.program_id(0)
    dv = dinv[...]
    t = dv * (q0[...] + q1[...]) + dv * dv * xp[...]
    z = jnp.dot(t, w[...], preferred_element_type=jnp.float32) + b[...]
    z_ref[...] = z

    @pl.when(i == 0)
    def _():
        acc[...] = jnp.zeros_like(acc)

    acc[0:1, :] += jnp.sum(z, axis=0, keepdims=True)
    acc[1:2, :] += jnp.sum(z * z, axis=0, keepdims=True)

    @pl.when(i == NB - 1)
    def _():
        m = acc[0:1, :] / N
        st_ref[0:1, :] = m
        st_ref[1:2, :] = acc[1:2, :] / N - m * m


def _mm_stats(q0, q1, xp, dinv, w, b):
    fi, fo = w.shape
    return pl.pallas_call(
        _mm_stats_body,
        grid=(NB,),
        in_specs=[
            pl.BlockSpec((BLK, fi), lambda i: (i, 0)),
            pl.BlockSpec((BLK, fi), lambda i: (i, 0)),
            pl.BlockSpec((BLK, fi), lambda i: (i, 0)),
            pl.BlockSpec((BLK, 1), lambda i: (i, 0)),
            pl.BlockSpec((fi, fo), lambda i: (0, 0)),
            pl.BlockSpec((1, fo), lambda i: (0, 0)),
        ],
        out_specs=[
            pl.BlockSpec((BLK, fo), lambda i: (i, 0)),
            pl.BlockSpec((2, fo), lambda i: (0, 0)),
        ],
        out_shape=[jax.ShapeDtypeStruct((N, fo), jnp.float32),
                   jax.ShapeDtypeStruct((2, fo), jnp.float32)],
        scratch_shapes=[pltpu.VMEM((2, fo), jnp.float32)],
    )(q0, q1, xp, dinv, w, b)


def _bn_relu_body(z_ref, st_ref, g_ref, be_ref, dinv_ref, x_ref, y_ref):
    mean = st_ref[0:1, :]
    var = st_ref[1:2, :]
    x = jnp.maximum((z_ref[...] - mean) * lax.rsqrt(var + 1e-5) * g_ref[...]
                    + be_ref[...], 0.0)
    x_ref[...] = x
    y_ref[...] = x * dinv_ref[...]


def _bn_relu(z, st, g, be, dinv):
    fo = z.shape[1]
    return pl.pallas_call(
        _bn_relu_body,
        grid=(NB,),
        in_specs=[
            pl.BlockSpec((BLK, fo), lambda i: (i, 0)),
            pl.BlockSpec((2, fo), lambda i: (0, 0)),
            pl.BlockSpec((1, fo), lambda i: (0, 0)),
            pl.BlockSpec((1, fo), lambda i: (0, 0)),
            pl.BlockSpec((BLK, 1), lambda i: (i, 0)),
        ],
        out_specs=[
            pl.BlockSpec((BLK, fo), lambda i: (i, 0)),
            pl.BlockSpec((BLK, fo), lambda i: (i, 0)),
        ],
        out_shape=[jax.ShapeDtypeStruct((N, fo), jnp.float32),
                   jax.ShapeDtypeStruct((N, fo), jnp.float32)],
    )(z, st, g, be, dinv)


def _bn2(h, w, b, g, be):
    z = jnp.dot(h, w, preferred_element_type=jnp.float32) + b
    m = jnp.sum(z, axis=0, keepdims=True) / G
    v = jnp.sum(z * z, axis=0, keepdims=True) / G - m * m
    return jnp.maximum((z - m) * lax.rsqrt(v + 1e-5) * g + be, 0.0)


def _pool_head_body(z_ref, st_ref, g_ref, be_ref, batch_ref,
                    fw0, fb0, fg0, fbe0, fw1, fb1, fg1, fbe1, fw2, fb2,
                    out_ref, pooled):
    i = pl.program_id(0)

    @pl.when(i == 0)
    def _():
        pooled[...] = jnp.full_like(pooled, -jnp.inf)

    mean = st_ref[0:1, :]
    var = st_ref[1:2, :]
    x = jnp.maximum((z_ref[...] - mean) * lax.rsqrt(var + 1e-5) * g_ref[...]
                    + be_ref[...], 0.0)
    bb = batch_ref[...]
    parts = []
    for g in range(G):
        xm = jnp.where(bb == g, x, -jnp.inf)
        parts.append(jnp.max(xm, axis=0, keepdims=True))
    pooled[...] = jnp.maximum(pooled[...], jnp.concatenate(parts, axis=0))

    @pl.when(i == NB - 1)
    def _():
        h = _bn2(pooled[...], fw0[...], fb0[...], fg0[...], fbe0[...])
        h = _bn2(h, fw1[...], fb1[...], fg1[...], fbe1[...])
        q = jnp.dot(h, fw2[...], preferred_element_type=jnp.float32) + fb2[...]
        mx = jnp.max(q, axis=1, keepdims=True)
        out_ref[...] = q - mx - jnp.log(jnp.sum(jnp.exp(q - mx), axis=1,
                                                keepdims=True))


def _pool_head(z, st, g, be, batch2, fw0, fb0, fg0, fbe0,
               fw1, fb1, fg1, fbe1, fw2, fb2):
    fo = z.shape[1]
    full = lambda a: pl.BlockSpec(a.shape, lambda i: tuple(0 for _ in a.shape))
    return pl.pallas_call(
        _pool_head_body,
        grid=(NB,),
        in_specs=[
            pl.BlockSpec((BLK, fo), lambda i: (i, 0)),
            pl.BlockSpec((2, fo), lambda i: (0, 0)),
            pl.BlockSpec((1, fo), lambda i: (0, 0)),
            pl.BlockSpec((1, fo), lambda i: (0, 0)),
            pl.BlockSpec((BLK, 1), lambda i: (i, 0)),
            full(fw0), full(fb0), full(fg0), full(fbe0),
            full(fw1), full(fb1), full(fg1), full(fbe1),
            full(fw2), full(fb2),
        ],
        out_specs=pl.BlockSpec((G, CLASSES), lambda i: (0, 0)),
        out_shape=jax.ShapeDtypeStruct((G, CLASSES), jnp.float32),
        scratch_shapes=[pltpu.VMEM((G, fo), jnp.float32)],
    )(z, st, g, be, batch2, fw0, fb0, fg0, fbe0, fw1, fb1, fg1, fbe1, fw2, fb2)


def kernel(pos, edge_index, batch,
           W0, b0, g0, be0, W1, b1, g1, be1, W2, b2, g2, be2,
           W3, b3, g3, be3, W4, b4, g4, be4,
           fW0, fb0, fg0, fbe0, fW1, fb1, fg1, fbe1, fW2, fb2):
    f32 = jnp.float32
    s = edge_index[0]
    d = edge_index[1]
    pad = E_PAD - E
    s2 = jnp.concatenate([s, jnp.zeros((pad,), jnp.int32)]).reshape(E_PAD // CHUNK, CHUNK)
    d2 = jnp.concatenate([d, jnp.full((pad,), N, jnp.int32)]).reshape(E_PAD // CHUNK, CHUNK)
    zeros16 = jnp.zeros((RPT, 16), f32)
    zeros128 = jnp.zeros((RPT, 128), f32)

    hist = _AGG[16](jnp.ones((N, 16), f32), s2, d2, zeros16)
    pos16 = jnp.pad(pos, ((0, 0), (0, 13)))
    dinv, y = _prep(hist[0, :N], hist[1, :N], pos16)

    def aggregate(y):
        f = y.shape[1]
        if f <= 128:
            a = _AGG[f](y, s2, d2, zeros16 if f == 16 else zeros128)
            return a[0, :N], a[1, :N]
        halves = [_AGG[128](y[:, k:k + 128], s2, d2, zeros128)
                  for k in range(0, f, 128)]
        q0 = jnp.concatenate([a[0, :N] for a in halves], axis=1)
        q1 = jnp.concatenate([a[1, :N] for a in halves], axis=1)
        return q0, q1

    x = pos16
    Ws = [jnp.pad(W0, ((0, 13), (0, 0))), W1, W2, W3, W4]
    bs = [b0, b1, b2, b3, b4]
    gs = [g0, g1, g2, g3, g4]
    bes = [be0, be1, be2, be3, be4]
    for i in range(5):
        q0, q1 = aggregate(y)
        z, st = _mm_stats(q0, q1, x, dinv, Ws[i], bs[i].reshape(1, -1))
        if i < 4:
            x, y = _bn_relu(z, st, gs[i].reshape(1, -1), bes[i].reshape(1, -1), dinv)
        else:
            return _pool_head(z, st, gs[i].reshape(1, -1), bes[i].reshape(1, -1),
                              batch.reshape(N, 1),
                              fW0, fb0.reshape(1, -1), fg0.reshape(1, -1),
                              fbe0.reshape(1, -1),
                              fW1, fb1.reshape(1, -1), fg1.reshape(1, -1),
                              fbe1.reshape(1, -1),
                              fW2, fb2.reshape(1, -1))


# speedup vs baseline: 5.2825x; 1.1385x over previous
import functools

import jax
import jax.numpy as jnp
from jax import lax
from jax.experimental import pallas as pl
from jax.experimental.pallas import tpu as pltpu
from jax.experimental.pallas import tpu_sc as plsc

N = 10000
E = 160000
G = 32
CLASSES = 40

NC = 2
NS = 16
NW = NC * NS
CHUNK = 128
J = 40
E_PAD = NW * J * CHUNK
N_PAD = NS * 632
RPT = N_PAD // NS
BLK = 1000
NB = N // BLK


def _make_agg_kernel(F):
    mesh = plsc.VectorSubcoreMesh(core_axis_name="c", subcore_axis_name="s",
                                  num_cores=NC, num_subcores=NS)
    NBUF = 4 if F <= 16 else 2
    NG = J // NBUF

    @functools.partial(
        pl.kernel, mesh=mesh,
        compiler_params=pltpu.CompilerParams(use_tc_tiling_on_sc=False),
        out_type=jax.ShapeDtypeStruct((NC, N_PAD, F), jnp.float32),
        scratch_types=[
            pltpu.VMEM((J, CHUNK), jnp.int32),
            pltpu.VMEM((J, CHUNK), jnp.int32),
            pltpu.VMEM((NBUF, CHUNK, F), jnp.float32),
            pltpu.VMEM_SHARED((N_PAD, F), jnp.float32),
            pltpu.SemaphoreType.DMA((NBUF,)),
            pltpu.SemaphoreType.DMA((NBUF,)),
            pltpu.SemaphoreType.DMA,
        ],
    )
    def agg(table_hbm, s_hbm, d_hbm, zeros_hbm, out_hbm,
            s_v, d_v, rows_v, acc_sh, gsem, ssem, zsem):
        cid = lax.axis_index("c")
        sid = lax.axis_index("s")
        wid = sid * NC + cid
        zc = pltpu.async_copy(zeros_hbm, acc_sh.at[pl.ds(sid * RPT, RPT)], zsem)
        pltpu.sync_copy(s_hbm.at[pl.ds(wid * J, J)], s_v)
        pltpu.sync_copy(d_hbm.at[pl.ds(wid * J, J)], d_v)
        zc.wait()
        plsc.subcore_barrier()

        def gissue(j, b):
            pltpu.async_copy(table_hbm.at[s_v.at[j]], rows_v.at[b], gsem.at[b])

        def gwait(b):
            pltpu.make_async_copy(table_hbm.at[s_v.at[0]], rows_v.at[b],
                                  gsem.at[b]).wait()

        def sissue(j, b):
            pltpu.async_copy(rows_v.at[b], acc_sh.at[d_v.at[j]], ssem.at[b],
                             add=True)

        def swait(b):
            pltpu.make_async_copy(rows_v.at[b], acc_sh.at[d_v.at[0]],
                                  ssem.at[b]).wait()

        for b in range(NBUF):
            gissue(b, b)

        def body(gi, carry):
            for b in range(NBUF):
                gwait(b)
                sissue(gi * NBUF + b, b)
                swait(b)
                gissue((gi + 1) * NBUF + b, b)
            return carry

        lax.fori_loop(0, NG - 1, body, 0)
        for b in range(NBUF):
            gwait(b)
            sissue((NG - 1) * NBUF + b, b)
            swait(b)
        plsc.subcore_barrier()
        pltpu.sync_copy(acc_sh.at[pl.ds(sid * RPT, RPT)],
                        out_hbm.at[cid, pl.ds(sid * RPT, RPT)])

    return agg


_AGG = {F: _make_agg_kernel(F) for F in (16, 128)}


def _prep_body(h0, h1, pos16, dinv_ref, y0_ref):
    deg = h0[:, 0:1] + h1[:, 0:1] + 1.0
    dv = lax.rsqrt(deg)
    dinv_ref[...] = dv
    y0_ref[...] = pos16[...] * dv


def _prep(h0, h1, pos16):
    return pl.pallas_call(
        _prep_body,
        out_shape=[jax.ShapeDtypeStruct((N, 1), jnp.float32),
                   jax.ShapeDtypeStruct((N, 16), jnp.float32)],
    )(h0, h1, pos16)


def _mm_stats_body(q0, q1, xp, dinv, w, b, z_ref, st_ref, acc):
    i = pl.program_id(0)
    dv = dinv[...]
    t = dv * (q0[...] + q1[...]) + dv * dv * xp[...]
    z = jnp.dot(t, w[...], preferred_element_type=jnp.float32) + b[...]
    z_ref[...] = z

    @pl.when(i == 0)
    def _():
        acc[...] = jnp.zeros_like(acc)

    acc[0:1, :] += jnp.sum(z, axis=0, keepdims=True)
    acc[1:2, :] += jnp.sum(z * z, axis=0, keepdims=True)

    @pl.when(i == NB - 1)
    def _():
        m = acc[0:1, :] / N
        st_ref[0:1, :] = m
        st_ref[1:2, :] = acc[1:2, :] / N - m * m


def _mm_stats(q0, q1, xp, dinv, w, b):
    fi, fo = w.shape
    return pl.pallas_call(
        _mm_stats_body,
        grid=(NB,),
        in_specs=[
            pl.BlockSpec((BLK, fi), lambda i: (i, 0)),
            pl.BlockSpec((BLK, fi), lambda i: (i, 0)),
            pl.BlockSpec((BLK, fi), lambda i: (i, 0)),
            pl.BlockSpec((BLK, 1), lambda i: (i, 0)),
            pl.BlockSpec((fi, fo), lambda i: (0, 0)),
            pl.BlockSpec((1, fo), lambda i: (0, 0)),
        ],
        out_specs=[
            pl.BlockSpec((BLK, fo), lambda i: (i, 0)),
            pl.BlockSpec((2, fo), lambda i: (0, 0)),
        ],
        out_shape=[jax.ShapeDtypeStruct((N, fo), jnp.float32),
                   jax.ShapeDtypeStruct((2, fo), jnp.float32)],
        scratch_shapes=[pltpu.VMEM((2, fo), jnp.float32)],
    )(q0, q1, xp, dinv, w, b)


def _bn_relu_body(z_ref, st_ref, g_ref, be_ref, dinv_ref, x_ref, y_ref):
    mean = st_ref[0:1, :]
    var = st_ref[1:2, :]
    x = jnp.maximum((z_ref[...] - mean) * lax.rsqrt(var + 1e-5) * g_ref[...]
                    + be_ref[...], 0.0)
    x_ref[...] = x
    y_ref[...] = x * dinv_ref[...]


def _bn_relu(z, st, g, be, dinv):
    fo = z.shape[1]
    return pl.pallas_call(
        _bn_relu_body,
        grid=(NB,),
        in_specs=[
            pl.BlockSpec((BLK, fo), lambda i: (i, 0)),
            pl.BlockSpec((2, fo), lambda i: (0, 0)),
            pl.BlockSpec((1, fo), lambda i: (0, 0)),
            pl.BlockSpec((1, fo), lambda i: (0, 0)),
            pl.BlockSpec((BLK, 1), lambda i: (i, 0)),
        ],
        out_specs=[
            pl.BlockSpec((BLK, fo), lambda i: (i, 0)),
            pl.BlockSpec((BLK, fo), lambda i: (i, 0)),
        ],
        out_shape=[jax.ShapeDtypeStruct((N, fo), jnp.float32),
                   jax.ShapeDtypeStruct((N, fo), jnp.float32)],
    )(z, st, g, be, dinv)


def _bn2(h, w, b, g, be):
    z = jnp.dot(h, w, preferred_element_type=jnp.float32) + b
    m = jnp.sum(z, axis=0, keepdims=True) / G
    v = jnp.sum(z * z, axis=0, keepdims=True) / G - m * m
    return jnp.maximum((z - m) * lax.rsqrt(v + 1e-5) * g + be, 0.0)


def _pool_head_body(z_ref, st_ref, g_ref, be_ref, batch_ref,
                    fw0, fb0, fg0, fbe0, fw1, fb1, fg1, fbe1, fw2, fb2,
                    out_ref, pooled):
    i = pl.program_id(0)

    @pl.when(i == 0)
    def _():
        pooled[...] = jnp.full_like(pooled, -jnp.inf)

    mean = st_ref[0:1, :]
    var = st_ref[1:2, :]
    x = jnp.maximum((z_ref[...] - mean) * lax.rsqrt(var + 1e-5) * g_ref[...]
                    + be_ref[...], 0.0)
    bb = batch_ref[...]
    parts = []
    for g in range(G):
        xm = jnp.where(bb == g, x, -jnp.inf)
        parts.append(jnp.max(xm, axis=0, keepdims=True))
    pooled[...] = jnp.maximum(pooled[...], jnp.concatenate(parts, axis=0))

    @pl.when(i == NB - 1)
    def _():
        h = _bn2(pooled[...], fw0[...], fb0[...], fg0[...], fbe0[...])
        h = _bn2(h, fw1[...], fb1[...], fg1[...], fbe1[...])
        q = jnp.dot(h, fw2[...], preferred_element_type=jnp.float32) + fb2[...]
        mx = jnp.max(q, axis=1, keepdims=True)
        out_ref[...] = q - mx - jnp.log(jnp.sum(jnp.exp(q - mx), axis=1,
                                                keepdims=True))


def _pool_head(z, st, g, be, batch2, fw0, fb0, fg0, fbe0,
               fw1, fb1, fg1, fbe1, fw2, fb2):
    fo = z.shape[1]
    full = lambda a: pl.BlockSpec(a.shape, lambda i: tuple(0 for _ in a.shape))
    return pl.pallas_call(
        _pool_head_body,
        grid=(NB,),
        in_specs=[
            pl.BlockSpec((BLK, fo), lambda i: (i, 0)),
            pl.BlockSpec((2, fo), lambda i: (0, 0)),
            pl.BlockSpec((1, fo), lambda i: (0, 0)),
            pl.BlockSpec((1, fo), lambda i: (0, 0)),
            pl.BlockSpec((BLK, 1), lambda i: (i, 0)),
            full(fw0), full(fb0), full(fg0), full(fbe0),
            full(fw1), full(fb1), full(fg1), full(fbe1),
            full(fw2), full(fb2),
        ],
        out_specs=pl.BlockSpec((G, CLASSES), lambda i: (0, 0)),
        out_shape=jax.ShapeDtypeStruct((G, CLASSES), jnp.float32),
        scratch_shapes=[pltpu.VMEM((G, fo), jnp.float32)],
    )(z, st, g, be, batch2, fw0, fb0, fg0, fbe0, fw1, fb1, fg1, fbe1, fw2, fb2)


def kernel(pos, edge_index, batch,
           W0, b0, g0, be0, W1, b1, g1, be1, W2, b2, g2, be2,
           W3, b3, g3, be3, W4, b4, g4, be4,
           fW0, fb0, fg0, fbe0, fW1, fb1, fg1, fbe1, fW2, fb2):
    f32 = jnp.float32
    s = edge_index[0]
    d = edge_index[1]
    pad = E_PAD - E
    s2 = jnp.concatenate([s, jnp.zeros((pad,), jnp.int32)]).reshape(E_PAD // CHUNK, CHUNK)
    d2 = jnp.concatenate([d, jnp.full((pad,), N, jnp.int32)]).reshape(E_PAD // CHUNK, CHUNK)
    zeros16 = jnp.zeros((RPT, 16), f32)
    zeros128 = jnp.zeros((RPT, 128), f32)

    hist = _AGG[16](jnp.ones((N, 16), f32), s2, d2, zeros16)
    pos16 = jnp.pad(pos, ((0, 0), (0, 13)))
    dinv, y = _prep(hist[0, :N], hist[1, :N], pos16)

    def aggregate(y):
        f = y.shape[1]
        if f <= 128:
            a = _AGG[f](y, s2, d2, zeros16 if f == 16 else zeros128)
            return a[0, :N], a[1, :N]
        halves = [_AGG[128](y[:, k:k + 128], s2, d2, zeros128)
                  for k in range(0, f, 128)]
        q0 = jnp.concatenate([a[0, :N] for a in halves], axis=1)
        q1 = jnp.concatenate([a[1, :N] for a in halves], axis=1)
        return q0, q1

    x = pos16
    Ws = [jnp.pad(W0, ((0, 13), (0, 0))), W1, W2, W3, W4]
    bs = [b0, b1, b2, b3, b4]
    gs = [g0, g1, g2, g3, g4]
    bes = [be0, be1, be2, be3, be4]
    for i in range(5):
        q0, q1 = aggregate(y)
        z, st = _mm_stats(q0, q1, x, dinv, Ws[i], bs[i].reshape(1, -1))
        if i < 4:
            x, y = _bn_relu(z, st, gs[i].reshape(1, -1), bes[i].reshape(1, -1), dinv)
        else:
            return _pool_head(z, st, gs[i].reshape(1, -1), bes[i].reshape(1, -1),
                              batch.reshape(N, 1),
                              fW0, fb0.reshape(1, -1), fg0.reshape(1, -1),
                              fbe0.reshape(1, -1),
                              fW1, fb1.reshape(1, -1), fg1.reshape(1, -1),
                              fbe1.reshape(1, -1),
                              fW2, fb2.reshape(1, -1))


# speedup vs baseline: 5.7865x; 1.0954x over previous
import functools

import jax
import jax.numpy as jnp
from jax import lax
from jax.experimental import pallas as pl
from jax.experimental.pallas import tpu as pltpu
from jax.experimental.pallas import tpu_sc as plsc

N = 10000
E = 160000
G = 32
CLASSES = 40

NC = 2
NS = 16
NW = NC * NS
CHUNK = 128
J = 40
E_PAD = NW * J * CHUNK
N_PAD = NS * 632
RPT = N_PAD // NS
BLK = 1000
NB = N // BLK


def _make_agg_kernel(F):
    mesh = plsc.VectorSubcoreMesh(core_axis_name="c", subcore_axis_name="s",
                                  num_cores=NC, num_subcores=NS)
    NBUF = 4 if F <= 16 else 2
    NG = J // NBUF

    @functools.partial(
        pl.kernel, mesh=mesh,
        compiler_params=pltpu.CompilerParams(use_tc_tiling_on_sc=False),
        out_type=jax.ShapeDtypeStruct((NC, N_PAD, F), jnp.float32),
        scratch_types=[
            pltpu.VMEM((J, CHUNK), jnp.int32),
            pltpu.VMEM((J, CHUNK), jnp.int32),
            pltpu.VMEM((NBUF, CHUNK, F), jnp.float32),
            pltpu.VMEM_SHARED((N_PAD, F), jnp.float32),
            pltpu.SemaphoreType.DMA((NBUF,)),
            pltpu.SemaphoreType.DMA((NBUF,)),
            pltpu.SemaphoreType.DMA,
        ],
    )
    def agg(table_hbm, s_hbm, d_hbm, zeros_hbm, out_hbm,
            s_v, d_v, rows_v, acc_sh, gsem, ssem, zsem):
        cid = lax.axis_index("c")
        sid = lax.axis_index("s")
        wid = sid * NC + cid
        zc = pltpu.async_copy(zeros_hbm, acc_sh.at[pl.ds(sid * RPT, RPT)], zsem)
        pltpu.sync_copy(s_hbm.at[pl.ds(wid * J, J)], s_v)
        pltpu.sync_copy(d_hbm.at[pl.ds(wid * J, J)], d_v)
        zc.wait()
        plsc.subcore_barrier()

        def gissue(j, b):
            pltpu.async_copy(table_hbm.at[s_v.at[j]], rows_v.at[b], gsem.at[b])

        def gwait(b):
            pltpu.make_async_copy(table_hbm.at[s_v.at[0]], rows_v.at[b],
                                  gsem.at[b]).wait()

        def sissue(j, b):
            pltpu.async_copy(rows_v.at[b], acc_sh.at[d_v.at[j]], ssem.at[b],
                             add=True)

        def swait(b):
            pltpu.make_async_copy(rows_v.at[b], acc_sh.at[d_v.at[0]],
                                  ssem.at[b]).wait()

        for b in range(NBUF):
            gissue(b, b)

        def body(gi, carry):
            for b in range(NBUF):
                gwait(b)
                sissue(gi * NBUF + b, b)
                swait(b)
                gissue((gi + 1) * NBUF + b, b)
            return carry

        lax.fori_loop(0, NG - 1, body, 0)
        for b in range(NBUF):
            gwait(b)
            sissue((NG - 1) * NBUF + b, b)
            swait(b)
        plsc.subcore_barrier()
        pltpu.sync_copy(acc_sh.at[pl.ds(sid * RPT, RPT)],
                        out_hbm.at[cid, pl.ds(sid * RPT, RPT)])

    return agg


_AGG = {F: _make_agg_kernel(F) for F in (16,)}

FH = 64
J2 = 80
PH = 2
JP = J2 // PH


def _make_split_kernel():
    mesh = plsc.VectorSubcoreMesh(core_axis_name="c", subcore_axis_name="s",
                                  num_cores=NC, num_subcores=NS)

    @functools.partial(
        pl.kernel, mesh=mesh,
        compiler_params=pltpu.CompilerParams(use_tc_tiling_on_sc=False),
        out_type=jax.ShapeDtypeStruct((NC, 2, N_PAD, FH), jnp.float32),
        scratch_types=[
            pltpu.VMEM((JP, CHUNK), jnp.int32),
            pltpu.VMEM((JP, CHUNK), jnp.int32),
            pltpu.VMEM((4, CHUNK, FH), jnp.float32),
            pltpu.VMEM_SHARED((N_PAD, FH), jnp.float32),
            pltpu.VMEM_SHARED((N_PAD, FH), jnp.float32),
            pltpu.SemaphoreType.DMA((4,)),
            pltpu.SemaphoreType.DMA((2,)),
            pltpu.SemaphoreType.DMA,
        ],
    )
    def agg(table_hbm, s_hbm, d_hbm, zeros_hbm, out_hbm,
            s_v, d_v, rows_v, acc_a, acc_b, gsem, ssem, zsem):
        cid = lax.axis_index("c")
        sid = lax.axis_index("s")
        za = pltpu.async_copy(zeros_hbm, acc_a.at[pl.ds(sid * RPT, RPT)], zsem)
        zb = pltpu.async_copy(zeros_hbm, acc_b.at[pl.ds(sid * RPT, RPT)], zsem)

        accs = (acc_a, acc_b)

        def gissue(j, b):
            pltpu.async_copy(table_hbm.at[s_v.at[j]], rows_v.at[b], gsem.at[b])

        def gwait(b):
            pltpu.make_async_copy(table_hbm.at[s_v.at[0]], rows_v.at[b],
                                  gsem.at[b]).wait()

        def sissue(j, b, par):
            pltpu.async_copy(rows_v.at[b], accs[par].at[d_v.at[j]],
                             ssem.at[par], add=True)

        def swait(par):
            pltpu.make_async_copy(rows_v.at[0], accs[par].at[d_v.at[0]],
                                  ssem.at[par]).wait()

        for p in range(PH):
            pltpu.sync_copy(s_hbm.at[cid, pl.ds(sid * J2 + p * JP, JP)], s_v)
            pltpu.sync_copy(d_hbm.at[pl.ds(sid * J2 + p * JP, JP)], d_v)
            if p == 0:
                za.wait()
                zb.wait()
                plsc.subcore_barrier()
            gissue(0, 0)
            gissue(1, 1)
            gwait(0)
            sissue(0, 0, 0)
            gissue(2, 2)
            gwait(1)
            sissue(1, 1, 1)
            gissue(3, 3)

            def body(gi, carry):
                for k in range(4):
                    j = gi * 4 + 2 + k
                    b = (2 + k) % 4
                    par = k % 2
                    gwait(b)
                    swait(par)
                    sissue(j, b, par)
                    gissue(j + 2, k)
                return carry

            lax.fori_loop(0, (JP - 4) // 4, body, 0)
            gwait(2)
            swait(0)
            sissue(JP - 2, 2, 0)
            gwait(3)
            swait(1)
            sissue(JP - 1, 3, 1)
            swait(0)
            swait(1)
        plsc.subcore_barrier()
        pltpu.sync_copy(acc_a.at[pl.ds(sid * RPT, RPT)],
                        out_hbm.at[cid, 0, pl.ds(sid * RPT, RPT)])
        pltpu.sync_copy(acc_b.at[pl.ds(sid * RPT, RPT)],
                        out_hbm.at[cid, 1, pl.ds(sid * RPT, RPT)])

    return agg


_AGG_SPLIT = _make_split_kernel()


def _prep_body(h0, h1, pos16, dinv_ref, y0_ref):
    deg = h0[:, 0:1] + h1[:, 0:1] + 1.0
    dv = lax.rsqrt(deg)
    dinv_ref[...] = dv
    y0_ref[...] = pos16[...] * dv


def _prep(h0, h1, pos16):
    return pl.pallas_call(
        _prep_body,
        out_shape=[jax.ShapeDtypeStruct((N, 1), jnp.float32),
                   jax.ShapeDtypeStruct((N, 16), jnp.float32)],
    )(h0, h1, pos16)


def _mm_stats_body(q0, q1, xp, dinv, w, b, z_ref, st_ref, acc):
    i = pl.program_id(0)
    dv = dinv[...]
    t = dv * (q0[...] + q1[...]) + dv * dv * xp[...]
    z = jnp.dot(t, w[...], preferred_element_type=jnp.float32) + b[...]
    z_ref[...] = z

    @pl.when(i == 0)
    def _():
        acc[...] = jnp.zeros_like(acc)

    acc[0:1, :] += jnp.sum(z, axis=0, keepdims=True)
    acc[1:2, :] += jnp.sum(z * z, axis=0, keepdims=True)

    @pl.when(i == NB - 1)
    def _():
        m = acc[0:1, :] / N
        st_ref[0:1, :] = m
        st_ref[1:2, :] = acc[1:2, :] / N - m * m


def _mm_stats(q0, q1, xp, dinv, w, b):
    fi, fo = w.shape
    return pl.pallas_call(
        _mm_stats_body,
        grid=(NB,),
        in_specs=[
            pl.BlockSpec((BLK, fi), lambda i: (i, 0)),
            pl.BlockSpec((BLK, fi), lambda i: (i, 0)),
            pl.BlockSpec((BLK, fi), lambda i: (i, 0)),
            pl.BlockSpec((BLK, 1), lambda i: (i, 0)),
            pl.BlockSpec((fi, fo), lambda i: (0, 0)),
            pl.BlockSpec((1, fo), lambda i: (0, 0)),
        ],
        out_specs=[
            pl.BlockSpec((BLK, fo), lambda i: (i, 0)),
            pl.BlockSpec((2, fo), lambda i: (0, 0)),
        ],
        out_shape=[jax.ShapeDtypeStruct((N, fo), jnp.float32),
                   jax.ShapeDtypeStruct((2, fo), jnp.float32)],
        scratch_shapes=[pltpu.VMEM((2, fo), jnp.float32)],
    )(q0, q1, xp, dinv, w, b)


def _bn_relu_body(z_ref, st_ref, g_ref, be_ref, dinv_ref, x_ref, y_ref):
    mean = st_ref[0:1, :]
    var = st_ref[1:2, :]
    x = jnp.maximum((z_ref[...] - mean) * lax.rsqrt(var + 1e-5) * g_ref[...]
                    + be_ref[...], 0.0)
    x_ref[...] = x
    y = x * dinv_ref[...]
    for k in range(y_ref.shape[0]):
        y_ref[k] = y[:, k * FH:(k + 1) * FH]


def _bn_relu(z, st, g, be, dinv):
    fo = z.shape[1]
    kp = fo // FH
    return pl.pallas_call(
        _bn_relu_body,
        grid=(NB,),
        in_specs=[
            pl.BlockSpec((BLK, fo), lambda i: (i, 0)),
            pl.BlockSpec((2, fo), lambda i: (0, 0)),
            pl.BlockSpec((1, fo), lambda i: (0, 0)),
            pl.BlockSpec((1, fo), lambda i: (0, 0)),
            pl.BlockSpec((BLK, 1), lambda i: (i, 0)),
        ],
        out_specs=[
            pl.BlockSpec((BLK, fo), lambda i: (i, 0)),
            pl.BlockSpec((kp, BLK, FH), lambda i: (0, i, 0)),
        ],
        out_shape=[jax.ShapeDtypeStruct((N, fo), jnp.float32),
                   jax.ShapeDtypeStruct((kp, N, FH), jnp.float32)],
    )(z, st, g, be, dinv)


def _mm_stats_split_body(*refs):
    qs = refs[:-6]
    xp, dinv, w, b, z_ref, st_ref = refs[-6:-1] + (refs[-1],)
    i = pl.program_id(0)
    dv = dinv[...]
    t = jnp.concatenate([qs[2 * k][...] + qs[2 * k + 1][...]
                         for k in range(len(qs) // 2)], axis=1)
    t = dv * t + dv * dv * xp[...]
    z = jnp.dot(t, w[...], preferred_element_type=jnp.float32) + b[...]
    z_ref[...] = z
    acc = st_ref
    @pl.when(i == 0)
    def _():
        acc[...] = jnp.zeros_like(acc)

    acc[0:1, :] += jnp.sum(z, axis=0, keepdims=True)
    acc[1:2, :] += jnp.sum(z * z, axis=0, keepdims=True)

    @pl.when(i == NB - 1)
    def _():
        m = acc[0:1, :] / N
        st_ref[0:1, :] = m
        st_ref[1:2, :] = acc[1:2, :] / N - m * m


def _mm_stats_split(qs, xp, dinv, w, b):
    fi, fo = w.shape
    qspec = [pl.BlockSpec((BLK, FH), lambda i: (i, 0)) for _ in qs]
    return pl.pallas_call(
        _mm_stats_split_body,
        grid=(NB,),
        in_specs=qspec + [
            pl.BlockSpec((BLK, fi), lambda i: (i, 0)),
            pl.BlockSpec((BLK, 1), lambda i: (i, 0)),
            pl.BlockSpec((fi, fo), lambda i: (0, 0)),
            pl.BlockSpec((1, fo), lambda i: (0, 0)),
        ],
        out_specs=[
            pl.BlockSpec((BLK, fo), lambda i: (i, 0)),
            pl.BlockSpec((2, fo), lambda i: (0, 0)),
        ],
        out_shape=[jax.ShapeDtypeStruct((N, fo), jnp.float32),
                   jax.ShapeDtypeStruct((2, fo), jnp.float32)],
    )(*qs, xp, dinv, w, b)


def _bn2(h, w, b, g, be):
    z = jnp.dot(h, w, preferred_element_type=jnp.float32) + b
    m = jnp.sum(z, axis=0, keepdims=True) / G
    v = jnp.sum(z * z, axis=0, keepdims=True) / G - m * m
    return jnp.maximum((z - m) * lax.rsqrt(v + 1e-5) * g + be, 0.0)


def _pool_head_body(z_ref, st_ref, g_ref, be_ref, batch_ref,
                    fw0, fb0, fg0, fbe0, fw1, fb1, fg1, fbe1, fw2, fb2,
                    out_ref, pooled):
    i = pl.program_id(0)

    @pl.when(i == 0)
    def _():
        pooled[...] = jnp.full_like(pooled, -jnp.inf)

    mean = st_ref[0:1, :]
    var = st_ref[1:2, :]
    x = jnp.maximum((z_ref[...] - mean) * lax.rsqrt(var + 1e-5) * g_ref[...]
                    + be_ref[...], 0.0)
    bb = batch_ref[...]
    parts = []
    for g in range(G):
        xm = jnp.where(bb == g, x, -jnp.inf)
        parts.append(jnp.max(xm, axis=0, keepdims=True))
    pooled[...] = jnp.maximum(pooled[...], jnp.concatenate(parts, axis=0))

    @pl.when(i == NB - 1)
    def _():
        h = _bn2(pooled[...], fw0[...], fb0[...], fg0[...], fbe0[...])
        h = _bn2(h, fw1[...], fb1[...], fg1[...], fbe1[...])
        q = jnp.dot(h, fw2[...], preferred_element_type=jnp.float32) + fb2[...]
        mx = jnp.max(q, axis=1, keepdims=True)
        out_ref[...] = q - mx - jnp.log(jnp.sum(jnp.exp(q - mx), axis=1,
                                                keepdims=True))


def _pool_head(z, st, g, be, batch2, fw0, fb0, fg0, fbe0,
               fw1, fb1, fg1, fbe1, fw2, fb2):
    fo = z.shape[1]
    full = lambda a: pl.BlockSpec(a.shape, lambda i: tuple(0 for _ in a.shape))
    return pl.pallas_call(
        _pool_head_body,
        grid=(NB,),
        in_specs=[
            pl.BlockSpec((BLK, fo), lambda i: (i, 0)),
            pl.BlockSpec((2, fo), lambda i: (0, 0)),
            pl.BlockSpec((1, fo), lambda i: (0, 0)),
            pl.BlockSpec((1, fo), lambda i: (0, 0)),
            pl.BlockSpec((BLK, 1), lambda i: (i, 0)),
            full(fw0), full(fb0), full(fg0), full(fbe0),
            full(fw1), full(fb1), full(fg1), full(fbe1),
            full(fw2), full(fb2),
        ],
        out_specs=pl.BlockSpec((G, CLASSES), lambda i: (0, 0)),
        out_shape=jax.ShapeDtypeStruct((G, CLASSES), jnp.float32),
        scratch_shapes=[pltpu.VMEM((G, fo), jnp.float32)],
    )(z, st, g, be, batch2, fw0, fb0, fg0, fbe0, fw1, fb1, fg1, fbe1, fw2, fb2)


def kernel(pos, edge_index, batch,
           W0, b0, g0, be0, W1, b1, g1, be1, W2, b2, g2, be2,
           W3, b3, g3, be3, W4, b4, g4, be4,
           fW0, fb0, fg0, fbe0, fW1, fb1, fg1, fbe1, fW2, fb2):
    f32 = jnp.float32
    s = edge_index[0]
    d = edge_index[1]
    pad = E_PAD - E
    s1 = jnp.concatenate([s, jnp.zeros((pad,), jnp.int32)])
    s2 = s1.reshape(E_PAD // CHUNK, CHUNK)
    s2b = jnp.stack([s2, s2 + N])
    d2 = jnp.concatenate([d, jnp.full((pad,), N, jnp.int32)]).reshape(E_PAD // CHUNK, CHUNK)
    zeros16 = jnp.zeros((RPT, 16), f32)
    zeros64 = jnp.zeros((RPT, FH), f32)

    hist = _AGG[16](jnp.ones((N, 16), f32), s2, d2, zeros16)
    pos16 = jnp.pad(pos, ((0, 0), (0, 13)))
    dinv, y = _prep(hist[0, :N], hist[1, :N], pos16)

    def aggregate_split(yp):
        qs = []
        for gidx in range(yp.shape[0] // 2):
            tab = yp[2 * gidx:2 * gidx + 2].reshape(2 * N, FH)
            a = _AGG_SPLIT(tab, s2b, d2, zeros64)
            for c in range(NC):
                qs.extend([a[c, 0, :N], a[c, 1, :N]])
        return qs

    x = pos16
    Ws = [jnp.pad(W0, ((0, 13), (0, 0))), W1, W2, W3, W4]
    bs = [b0, b1, b2, b3, b4]
    gs = [g0, g1, g2, g3, g4]
    bes = [be0, be1, be2, be3, be4]
    for i in range(5):
        if i == 0:
            a = _AGG[16](y, s2, d2, zeros16)
            z, st = _mm_stats(a[0, :N], a[1, :N], x, dinv, Ws[i],
                              bs[i].reshape(1, -1))
        else:
            qs = aggregate_split(y)
            z, st = _mm_stats_split(qs, x, dinv, Ws[i], bs[i].reshape(1, -1))
        if i < 4:
            x, y = _bn_relu(z, st, gs[i].reshape(1, -1), bes[i].reshape(1, -1), dinv)
        else:
            return _pool_head(z, st, gs[i].reshape(1, -1), bes[i].reshape(1, -1),
                              batch.reshape(N, 1),
                              fW0, fb0.reshape(1, -1), fg0.reshape(1, -1),
                              fbe0.reshape(1, -1),
                              fW1, fb1.reshape(1, -1), fg1.reshape(1, -1),
                              fbe1.reshape(1, -1),
                              fW2, fb2.reshape(1, -1))


# speedup vs baseline: 6.1139x; 1.0566x over previous
import functools

import jax
import jax.numpy as jnp
from jax import lax
from jax.experimental import pallas as pl
from jax.experimental.pallas import tpu as pltpu
from jax.experimental.pallas import tpu_sc as plsc

N = 10000
E = 160000
G = 32
CLASSES = 40

NC = 2
NS = 16
NW = NC * NS
CHUNK = 128
J = 40
E_PAD = NW * J * CHUNK
N_PAD = NS * 632
RPT = N_PAD // NS
BLK = 1000
NB = N // BLK


def _make_agg_kernel(F):
    mesh = plsc.VectorSubcoreMesh(core_axis_name="c", subcore_axis_name="s",
                                  num_cores=NC, num_subcores=NS)
    NBUF = 4 if F <= 16 else 2
    NG = J // NBUF

    @functools.partial(
        pl.kernel, mesh=mesh,
        compiler_params=pltpu.CompilerParams(use_tc_tiling_on_sc=False),
        out_type=jax.ShapeDtypeStruct((NC, N_PAD, F), jnp.float32),
        scratch_types=[
            pltpu.VMEM((J, CHUNK), jnp.int32),
            pltpu.VMEM((J, CHUNK), jnp.int32),
            pltpu.VMEM((NBUF, CHUNK, F), jnp.float32),
            pltpu.VMEM_SHARED((N_PAD, F), jnp.float32),
            pltpu.SemaphoreType.DMA((NBUF,)),
            pltpu.SemaphoreType.DMA((NBUF,)),
            pltpu.SemaphoreType.DMA,
        ],
    )
    def agg(table_hbm, s_hbm, d_hbm, zeros_hbm, out_hbm,
            s_v, d_v, rows_v, acc_sh, gsem, ssem, zsem):
        cid = lax.axis_index("c")
        sid = lax.axis_index("s")
        wid = sid * NC + cid
        zc = pltpu.async_copy(zeros_hbm, acc_sh.at[pl.ds(sid * RPT, RPT)], zsem)
        pltpu.sync_copy(s_hbm.at[pl.ds(wid * J, J)], s_v)
        pltpu.sync_copy(d_hbm.at[pl.ds(wid * J, J)], d_v)
        zc.wait()
        plsc.subcore_barrier()

        def gissue(j, b):
            pltpu.async_copy(table_hbm.at[s_v.at[j]], rows_v.at[b], gsem.at[b])

        def gwait(b):
            pltpu.make_async_copy(table_hbm.at[s_v.at[0]], rows_v.at[b],
                                  gsem.at[b]).wait()

        def sissue(j, b):
            pltpu.async_copy(rows_v.at[b], acc_sh.at[d_v.at[j]], ssem.at[b],
                             add=True)

        def swait(b):
            pltpu.make_async_copy(rows_v.at[b], acc_sh.at[d_v.at[0]],
                                  ssem.at[b]).wait()

        for b in range(NBUF):
            gissue(b, b)

        def body(gi, carry):
            for b in range(NBUF):
                gwait(b)
                sissue(gi * NBUF + b, b)
                swait(b)
                gissue((gi + 1) * NBUF + b, b)
            return carry

        lax.fori_loop(0, NG - 1, body, 0)
        for b in range(NBUF):
            gwait(b)
            sissue((NG - 1) * NBUF + b, b)
            swait(b)
        plsc.subcore_barrier()
        pltpu.sync_copy(acc_sh.at[pl.ds(sid * RPT, RPT)],
                        out_hbm.at[cid, pl.ds(sid * RPT, RPT)])

    return agg


_AGG = {F: _make_agg_kernel(F) for F in (16,)}

FH = 64
J2 = 80
PH = 2
JP = J2 // PH


def _make_split_kernel():
    mesh = plsc.VectorSubcoreMesh(core_axis_name="c", subcore_axis_name="s",
                                  num_cores=NC, num_subcores=NS)

    @functools.partial(
        pl.kernel, mesh=mesh,
        compiler_params=pltpu.CompilerParams(use_tc_tiling_on_sc=False),
        out_type=jax.ShapeDtypeStruct((NC, 2, N_PAD, FH), jnp.float32),
        scratch_types=[
            pltpu.VMEM((JP, CHUNK), jnp.int32),
            pltpu.VMEM((JP, CHUNK), jnp.int32),
            pltpu.VMEM((4, CHUNK, FH), jnp.float32),
            pltpu.VMEM_SHARED((N_PAD, FH), jnp.float32),
            pltpu.VMEM_SHARED((N_PAD, FH), jnp.float32),
            pltpu.SemaphoreType.DMA((4,)),
            pltpu.SemaphoreType.DMA((2,)),
            pltpu.SemaphoreType.DMA,
        ],
    )
    def agg(table_hbm, s_hbm, d_hbm, zeros_hbm, out_hbm,
            s_v, d_v, rows_v, acc_a, acc_b, gsem, ssem, zsem):
        cid = lax.axis_index("c")
        sid = lax.axis_index("s")
        za = pltpu.async_copy(zeros_hbm, acc_a.at[pl.ds(sid * RPT, RPT)], zsem)
        zb = pltpu.async_copy(zeros_hbm, acc_b.at[pl.ds(sid * RPT, RPT)], zsem)

        accs = (acc_a, acc_b)

        def gissue(j, b):
            pltpu.async_copy(table_hbm.at[s_v.at[j]], rows_v.at[b], gsem.at[b])

        def gwait(b):
            pltpu.make_async_copy(table_hbm.at[s_v.at[0]], rows_v.at[b],
                                  gsem.at[b]).wait()

        def sissue(j, b, par):
            pltpu.async_copy(rows_v.at[b], accs[par].at[d_v.at[j]],
                             ssem.at[par], add=True)

        def swait(par):
            pltpu.make_async_copy(rows_v.at[0], accs[par].at[d_v.at[0]],
                                  ssem.at[par]).wait()

        for p in range(PH):
            pltpu.sync_copy(s_hbm.at[cid, pl.ds(sid * J2 + p * JP, JP)], s_v)
            pltpu.sync_copy(d_hbm.at[pl.ds(sid * J2 + p * JP, JP)], d_v)
            if p == 0:
                za.wait()
                zb.wait()
                plsc.subcore_barrier()
            gissue(0, 0)
            gissue(1, 1)
            gwait(0)
            sissue(0, 0, 0)
            gissue(2, 2)
            gwait(1)
            sissue(1, 1, 1)
            gissue(3, 3)

            def body(gi, carry):
                for k in range(4):
                    j = gi * 4 + 2 + k
                    b = (2 + k) % 4
                    par = k % 2
                    gwait(b)
                    swait(par)
                    sissue(j, b, par)
                    gissue(j + 2, k)
                return carry

            lax.fori_loop(0, (JP - 4) // 4, body, 0)
            gwait(2)
            swait(0)
            sissue(JP - 2, 2, 0)
            gwait(3)
            swait(1)
            sissue(JP - 1, 3, 1)
            swait(0)
            swait(1)
        plsc.subcore_barrier()
        pltpu.sync_copy(acc_a.at[pl.ds(sid * RPT, RPT)],
                        out_hbm.at[cid, 0, pl.ds(sid * RPT, RPT)])
        pltpu.sync_copy(acc_b.at[pl.ds(sid * RPT, RPT)],
                        out_hbm.at[cid, 1, pl.ds(sid * RPT, RPT)])

    return agg


_AGG_SPLIT = _make_split_kernel()


def _prep_body(h0, h1, pos16, dinv_ref, y0_ref):
    deg = h0[:, 0:1] + h1[:, 0:1] + 1.0
    dv = lax.rsqrt(deg)
    dinv_ref[...] = dv
    y0_ref[...] = pos16[...] * dv


def _prep(h0, h1, pos16):
    return pl.pallas_call(
        _prep_body,
        out_shape=[jax.ShapeDtypeStruct((N, 1), jnp.float32),
                   jax.ShapeDtypeStruct((N, 16), jnp.float32)],
    )(h0, h1, pos16)


def _mm_stats_body(q0, q1, xp, dinv, w, b, z_ref, st_ref, acc):
    i = pl.program_id(0)
    dv = dinv[...]
    t = dv * (q0[...] + q1[...]) + dv * dv * xp[...]
    z = jnp.dot(t, w[...], preferred_element_type=jnp.float32) + b[...]
    z_ref[...] = z

    @pl.when(i == 0)
    def _():
        acc[...] = jnp.zeros_like(acc)

    acc[0:1, :] += jnp.sum(z, axis=0, keepdims=True)
    acc[1:2, :] += jnp.sum(z * z, axis=0, keepdims=True)

    @pl.when(i == NB - 1)
    def _():
        m = acc[0:1, :] / N
        st_ref[0:1, :] = m
        st_ref[1:2, :] = acc[1:2, :] / N - m * m


def _mm_stats(q0, q1, xp, dinv, w, b):
    fi, fo = w.shape
    return pl.pallas_call(
        _mm_stats_body,
        grid=(NB,),
        in_specs=[
            pl.BlockSpec((BLK, fi), lambda i: (i, 0)),
            pl.BlockSpec((BLK, fi), lambda i: (i, 0)),
            pl.BlockSpec((BLK, fi), lambda i: (i, 0)),
            pl.BlockSpec((BLK, 1), lambda i: (i, 0)),
            pl.BlockSpec((fi, fo), lambda i: (0, 0)),
            pl.BlockSpec((1, fo), lambda i: (0, 0)),
        ],
        out_specs=[
            pl.BlockSpec((BLK, fo), lambda i: (i, 0)),
            pl.BlockSpec((2, fo), lambda i: (0, 0)),
        ],
        out_shape=[jax.ShapeDtypeStruct((N, fo), jnp.float32),
                   jax.ShapeDtypeStruct((2, fo), jnp.float32)],
        scratch_shapes=[pltpu.VMEM((2, fo), jnp.float32)],
    )(q0, q1, xp, dinv, w, b)


def _bn_relu_body(z_ref, st_ref, g_ref, be_ref, dinv_ref, x_ref, y_ref):
    mean = st_ref[0:1, :]
    var = st_ref[1:2, :]
    x = jnp.maximum((z_ref[...] - mean) * lax.rsqrt(var + 1e-5) * g_ref[...]
                    + be_ref[...], 0.0)
    x_ref[...] = x
    y = x * dinv_ref[...]
    for k in range(y_ref.shape[0]):
        y_ref[k] = y[:, k * FH:(k + 1) * FH]


def _bn_relu(z, st, g, be, dinv):
    fo = z.shape[1]
    kp = fo // FH
    return pl.pallas_call(
        _bn_relu_body,
        grid=(NB,),
        in_specs=[
            pl.BlockSpec((BLK, fo), lambda i: (i, 0)),
            pl.BlockSpec((2, fo), lambda i: (0, 0)),
            pl.BlockSpec((1, fo), lambda i: (0, 0)),
            pl.BlockSpec((1, fo), lambda i: (0, 0)),
            pl.BlockSpec((BLK, 1), lambda i: (i, 0)),
        ],
        out_specs=[
            pl.BlockSpec((BLK, fo), lambda i: (i, 0)),
            pl.BlockSpec((kp, BLK, FH), lambda i: (0, i, 0)),
        ],
        out_shape=[jax.ShapeDtypeStruct((N, fo), jnp.float32),
                   jax.ShapeDtypeStruct((kp, N, FH), jnp.float32)],
    )(z, st, g, be, dinv)


def _mm_stats_split_body(*refs):
    qs = refs[:-6]
    xp, dinv, w, b, z_ref, st_ref = refs[-6:-1] + (refs[-1],)
    i = pl.program_id(0)
    dv = dinv[...]
    t = jnp.concatenate([qs[2 * k][0, 0] + qs[2 * k + 1][0, 0]
                         for k in range(len(qs) // 2)], axis=1)
    t = dv * t + dv * dv * xp[...]
    z = jnp.dot(t, w[...], preferred_element_type=jnp.float32) + b[...]
    z_ref[...] = z
    acc = st_ref
    @pl.when(i == 0)
    def _():
        acc[...] = jnp.zeros_like(acc)

    acc[0:1, :] += jnp.sum(z, axis=0, keepdims=True)
    acc[1:2, :] += jnp.sum(z * z, axis=0, keepdims=True)

    @pl.when(i == NB - 1)
    def _():
        m = acc[0:1, :] / N
        st_ref[0:1, :] = m
        st_ref[1:2, :] = acc[1:2, :] / N - m * m


def _mm_stats_split(arrs, xp, dinv, w, b):
    fi, fo = w.shape
    qs = []
    qspec = []
    for a in arrs:
        for c in range(NC):
            for p in range(2):
                qs.append(a)
                qspec.append(pl.BlockSpec(
                    (1, 1, BLK, FH), lambda i, c=c, p=p: (c, p, i, 0)))
    return pl.pallas_call(
        _mm_stats_split_body,
        grid=(NB,),
        in_specs=qspec + [
            pl.BlockSpec((BLK, fi), lambda i: (i, 0)),
            pl.BlockSpec((BLK, 1), lambda i: (i, 0)),
            pl.BlockSpec((fi, fo), lambda i: (0, 0)),
            pl.BlockSpec((1, fo), lambda i: (0, 0)),
        ],
        out_specs=[
            pl.BlockSpec((BLK, fo), lambda i: (i, 0)),
            pl.BlockSpec((2, fo), lambda i: (0, 0)),
        ],
        out_shape=[jax.ShapeDtypeStruct((N, fo), jnp.float32),
                   jax.ShapeDtypeStruct((2, fo), jnp.float32)],
    )(*qs, xp, dinv, w, b)


def _bn2(h, w, b, g, be):
    z = jnp.dot(h, w, preferred_element_type=jnp.float32) + b
    m = jnp.sum(z, axis=0, keepdims=True) / G
    v = jnp.sum(z * z, axis=0, keepdims=True) / G - m * m
    return jnp.maximum((z - m) * lax.rsqrt(v + 1e-5) * g + be, 0.0)


def _pool_head_body(z_ref, st_ref, g_ref, be_ref, batch_ref,
                    fw0, fb0, fg0, fbe0, fw1, fb1, fg1, fbe1, fw2, fb2,
                    out_ref, pooled):
    i = pl.program_id(0)

    @pl.when(i == 0)
    def _():
        pooled[...] = jnp.full_like(pooled, -jnp.inf)

    mean = st_ref[0:1, :]
    var = st_ref[1:2, :]
    x = jnp.maximum((z_ref[...] - mean) * lax.rsqrt(var + 1e-5) * g_ref[...]
                    + be_ref[...], 0.0)
    bb = batch_ref[...]
    parts = []
    for g in range(G):
        xm = jnp.where(bb == g, x, -jnp.inf)
        parts.append(jnp.max(xm, axis=0, keepdims=True))
    pooled[...] = jnp.maximum(pooled[...], jnp.concatenate(parts, axis=0))

    @pl.when(i == NB - 1)
    def _():
        h = _bn2(pooled[...], fw0[...], fb0[...], fg0[...], fbe0[...])
        h = _bn2(h, fw1[...], fb1[...], fg1[...], fbe1[...])
        q = jnp.dot(h, fw2[...], preferred_element_type=jnp.float32) + fb2[...]
        mx = jnp.max(q, axis=1, keepdims=True)
        out_ref[...] = q - mx - jnp.log(jnp.sum(jnp.exp(q - mx), axis=1,
                                                keepdims=True))


def _pool_head(z, st, g, be, batch2, fw0, fb0, fg0, fbe0,
               fw1, fb1, fg1, fbe1, fw2, fb2):
    fo = z.shape[1]
    full = lambda a: pl.BlockSpec(a.shape, lambda i: tuple(0 for _ in a.shape))
    return pl.pallas_call(
        _pool_head_body,
        grid=(NB,),
        in_specs=[
            pl.BlockSpec((BLK, fo), lambda i: (i, 0)),
            pl.BlockSpec((2, fo), lambda i: (0, 0)),
            pl.BlockSpec((1, fo), lambda i: (0, 0)),
            pl.BlockSpec((1, fo), lambda i: (0, 0)),
            pl.BlockSpec((BLK, 1), lambda i: (i, 0)),
            full(fw0), full(fb0), full(fg0), full(fbe0),
            full(fw1), full(fb1), full(fg1), full(fbe1),
            full(fw2), full(fb2),
        ],
        out_specs=pl.BlockSpec((G, CLASSES), lambda i: (0, 0)),
        out_shape=jax.ShapeDtypeStruct((G, CLASSES), jnp.float32),
        scratch_shapes=[pltpu.VMEM((G, fo), jnp.float32)],
    )(z, st, g, be, batch2, fw0, fb0, fg0, fbe0, fw1, fb1, fg1, fbe1, fw2, fb2)


def kernel(pos, edge_index, batch,
           W0, b0, g0, be0, W1, b1, g1, be1, W2, b2, g2, be2,
           W3, b3, g3, be3, W4, b4, g4, be4,
           fW0, fb0, fg0, fbe0, fW1, fb1, fg1, fbe1, fW2, fb2):
    f32 = jnp.float32
    s = edge_index[0]
    d = edge_index[1]
    pad = E_PAD - E
    s1 = jnp.concatenate([s, jnp.zeros((pad,), jnp.int32)])
    s2 = s1.reshape(E_PAD // CHUNK, CHUNK)
    s2b = jnp.stack([s2, s2 + N])
    d2 = jnp.concatenate([d, jnp.full((pad,), N, jnp.int32)]).reshape(E_PAD // CHUNK, CHUNK)
    zeros16 = jnp.zeros((RPT, 16), f32)
    zeros64 = jnp.zeros((RPT, FH), f32)

    hist = _AGG[16](jnp.ones((N, 16), f32), s2, d2, zeros16)
    pos16 = jnp.pad(pos, ((0, 0), (0, 13)))
    dinv, y = _prep(hist[0, :N], hist[1, :N], pos16)

    def aggregate_split(yp):
        return [_AGG_SPLIT(yp[2 * g:2 * g + 2].reshape(2 * N, FH),
                           s2b, d2, zeros64)
                for g in range(yp.shape[0] // 2)]

    x = pos16
    Ws = [jnp.pad(W0, ((0, 13), (0, 0))), W1, W2, W3, W4]
    bs = [b0, b1, b2, b3, b4]
    gs = [g0, g1, g2, g3, g4]
    bes = [be0, be1, be2, be3, be4]
    for i in range(5):
        if i == 0:
            a = _AGG[16](y, s2, d2, zeros16)
            z, st = _mm_stats(a[0, :N], a[1, :N], x, dinv, Ws[i],
                              bs[i].reshape(1, -1))
        else:
            qs = aggregate_split(y)
            z, st = _mm_stats_split(qs, x, dinv, Ws[i], bs[i].reshape(1, -1))
        if i < 4:
            x, y = _bn_relu(z, st, gs[i].reshape(1, -1), bes[i].reshape(1, -1), dinv)
        else:
            return _pool_head(z, st, gs[i].reshape(1, -1), bes[i].reshape(1, -1),
                              batch.reshape(N, 1),
                              fW0, fb0.reshape(1, -1), fg0.reshape(1, -1),
                              fbe0.reshape(1, -1),
                              fW1, fb1.reshape(1, -1), fg1.reshape(1, -1),
                              fbe1.reshape(1, -1),
                              fW2, fb2.reshape(1, -1))
